# Initial kernel scaffold; baseline (speedup 1.0000x reference)
import jax
import jax.numpy as jnp
from jax.experimental import pallas as pl


def kernel(ner_ids, dep_ids, p_ner_ids, ner_table, dep_table, p_ner_table):
    raise NotImplementedError("write your pallas kernel here")


# speedup vs baseline: 4.1705x; 4.1705x over previous
import functools

import jax
import jax.numpy as jnp
from jax import lax
from jax.experimental import pallas as pl
from jax.experimental.pallas import tpu as pltpu
from jax.experimental.pallas import tpu_sc as plsc

_EMBED = 64
_OUT_D = 192
_PAD_D = 128
_NC, _NS = 2, 16
_NW = _NC * _NS
_CHUNK = 128


def _build(n_tokens: int):
    n_per_w = n_tokens // _NW
    n_chunks = n_per_w // _CHUNK
    assert n_per_w % _CHUNK == 0

    mesh = plsc.VectorSubcoreMesh(
        core_axis_name="c", subcore_axis_name="s",
        num_cores=_NC, num_subcores=_NS,
    )

    @functools.partial(
        pl.kernel,
        out_type=jax.ShapeDtypeStruct((n_tokens, _OUT_D), jnp.float32),
        mesh=mesh,
        scratch_types=[
            pltpu.VMEM((3 * _CHUNK,), jnp.int32),
            pltpu.VMEM((_CHUNK, _PAD_D), jnp.float32),
            pltpu.VMEM((_CHUNK, _PAD_D), jnp.float32),
            pltpu.VMEM((_CHUNK, _OUT_D), jnp.float32),
            pltpu.SemaphoreType.DMA,
        ],
    )
    def run(ner_i, dep_i, pner_i, ner_t, dep_t, pner_t, out,
            idx_v, dep_v, pner_v, big_v, sem):
        wid = lax.axis_index("s") * _NC + lax.axis_index("c")
        base_w = wid * n_per_w

        def body(ci, _):
            base = base_w + ci * _CHUNK
            pltpu.sync_copy(ner_i.at[pl.ds(base, _CHUNK)],
                            idx_v.at[pl.ds(0, _CHUNK)])
            pltpu.sync_copy(dep_i.at[pl.ds(base, _CHUNK)],
                            idx_v.at[pl.ds(_CHUNK, _CHUNK)])
            pltpu.sync_copy(pner_i.at[pl.ds(base, _CHUNK)],
                            idx_v.at[pl.ds(2 * _CHUNK, _CHUNK)])
            c0 = pltpu.async_copy(
                ner_t.at[idx_v.at[pl.ds(0, _CHUNK)]],
                big_v.at[:, pl.ds(0, _PAD_D)], sem)
            c1 = pltpu.async_copy(
                dep_t.at[idx_v.at[pl.ds(_CHUNK, _CHUNK)]], dep_v, sem)
            c2 = pltpu.async_copy(
                pner_t.at[idx_v.at[pl.ds(2 * _CHUNK, _CHUNK)]], pner_v, sem)
            c0.wait()
            c1.wait()
            c2.wait()

            def interleave(i, _):
                for m in range(_EMBED // 16):
                    big_v[i, pl.ds(_EMBED + m * 16, 16)] = (
                        dep_v[i, pl.ds(m * 16, 16)])
                    big_v[i, pl.ds(2 * _EMBED + m * 16, 16)] = (
                        pner_v[i, pl.ds(m * 16, 16)])
                return 0

            lax.fori_loop(0, _CHUNK, interleave, 0)
            pltpu.sync_copy(big_v, out.at[pl.ds(base, _CHUNK)])
            return 0

        lax.fori_loop(0, n_chunks, body, 0)

    return run


def kernel(ner_ids, dep_ids, p_ner_ids, ner_table, dep_table, p_ner_table):
    B, L = ner_ids.shape
    n_tokens = B * L
    ids = [a.reshape(n_tokens).astype(jnp.int32)
           for a in (ner_ids, dep_ids, p_ner_ids)]
    pad = ((0, 0), (0, _PAD_D - _EMBED))
    tabs = [jnp.pad(t, pad) for t in (ner_table, dep_table, p_ner_table)]
    run = _build(n_tokens)
    out = run(*ids, *tabs)
    return out.reshape(B, L, _OUT_D)
